# baseline (speedup 1.0000x reference)
import functools

import jax
import jax.numpy as jnp
from jax import lax
from jax.experimental import pallas as pl
from jax.experimental.pallas import tpu as pltpu

N_TOTAL = 32768
B = 256
NUM_NODES = 128
D = 128
H = 64
F32 = jnp.float32

_INTERPRET = False


def _dense_body(idx_ref, hist_in_ref, hist_out_ref, emb_ref, Win_ref, Wout_ref,
                Wr_ref, BD1_ref, b1_ref, BD2_ref, b2_ref, bias_ref, ng_ref,
                nb_ref, We1_ref, be1_ref, eg_ref, eb_ref, We2r_ref, be2_ref,
                dots_ref, exit_ref, S_scr):
    b = pl.program_id(0)

    @pl.when(b == 0)
    def _():
        e = emb_ref[...]
        S_scr[0:D] = jnp.dot(e, Win_ref[...], preferred_element_type=F32, precision=lax.Precision.HIGHEST)
        S_scr[D:2 * D] = jnp.dot(e, Wout_ref[...], preferred_element_type=F32, precision=lax.Precision.HIGHEST)
        S_scr[2 * D:3 * D] = jnp.dot(e, Wr_ref[...], preferred_element_type=F32, precision=lax.Precision.HIGHEST)

    idxv = idx_ref[0]
    k_iota = lax.broadcasted_iota(jnp.int32, (D, NUM_NODES), 0)
    onehotT = (k_iota == idxv).astype(F32)

    xn = (jnp.dot(hist_in_ref[...], S_scr[0:D], preferred_element_type=F32, precision=lax.Precision.HIGHEST)
          + jnp.dot(hist_out_ref[...], S_scr[D:2 * D], preferred_element_type=F32, precision=lax.Precision.HIGHEST)
          + lax.dot_general(onehotT, S_scr[2 * D:3 * D],
                            (((0,), (0,)), ((), ())), preferred_element_type=F32, precision=lax.Precision.HIGHEST)
          + bias_ref[...])

    h1 = jnp.maximum(jnp.dot(xn, BD1_ref[...], preferred_element_type=F32, precision=lax.Precision.HIGHEST)
                     + b1_ref[...], 0.0)
    x = jnp.dot(h1, BD2_ref[...], preferred_element_type=F32, precision=lax.Precision.HIGHEST) + b2_ref[...]

    mu = jnp.mean(x, axis=1, keepdims=True)
    xc = x - mu
    var = jnp.mean(xc * xc, axis=1, keepdims=True)
    xln = xc * lax.rsqrt(var + 1e-5) * ng_ref[...] + nb_ref[...]

    m = jnp.mean(xln, axis=0, keepdims=True)
    hm = jnp.maximum(jnp.dot(m, We1_ref[...], preferred_element_type=F32, precision=lax.Precision.HIGHEST)
                     + be1_ref[...], 0.0)
    mu2 = jnp.mean(hm, axis=1, keepdims=True)
    hc = hm - mu2
    var2 = jnp.mean(hc * hc, axis=1, keepdims=True)
    hln = hc * lax.rsqrt(var2 + 1e-5) * eg_ref[...] + eb_ref[...]
    exit_ref[pl.ds(b, 1), :] = (jnp.sum(hln * We2r_ref[...], axis=1,
                                        keepdims=True) + be2_ref[...])

    a = xln[:, :H]
    c2 = xln[:, H:]
    dots_ref[...] = lax.dot_general(
        a, c2, (((1,), (1,)), ((), ())), preferred_element_type=F32,
        precision=lax.Precision.HIGHEST) * (1.0 / 8.0)


def _dense_pipeline(idx, hist_in, hist_out, emb, Win, Wout, Wr, BD1, b1, BD2,
                    b2, bias_comb, ng, nb, We1, be1, eg, eb, We2r, be2):
    full = lambda s: pl.BlockSpec(s, lambda b: tuple(0 for _ in s))
    grid = (B,)
    dots, exitv = pl.pallas_call(
        _dense_body,
        grid=grid,
        in_specs=[
            pl.BlockSpec((1, 1, NUM_NODES), lambda b: (b, 0, 0)),
            pl.BlockSpec((NUM_NODES, D), lambda b: (b, 0)),
            pl.BlockSpec((NUM_NODES, D), lambda b: (b, 0)),
            full((D, D)), full((D, D)), full((D, D)), full((D, D)),
            full((D, D)), full((1, D)), full((D, D)), full((1, D)),
            full((1, D)), full((1, D)), full((1, D)),
            full((D, D)), full((1, D)), full((1, D)), full((1, D)),
            full((1, D)), full((1, 1)),
        ],
        out_specs=[
            pl.BlockSpec((NUM_NODES, D), lambda b: (b, 0)),
            pl.BlockSpec((B, 1), lambda b: (0, 0)),
        ],
        out_shape=[
            jax.ShapeDtypeStruct((N_TOTAL, D), F32),
            jax.ShapeDtypeStruct((B, 1), F32),
        ],
        scratch_shapes=[pltpu.VMEM((3 * D, D), F32)],
        interpret=_INTERPRET,
    )(idx, hist_in, hist_out, emb, Win, Wout, Wr, BD1, b1, BD2, b2, bias_comb,
      ng, nb, We1, be1, eg, eb, We2r, be2)
    return dots, exitv


def kernel(node_features, edge_index, ptr, emb, Win, bin_, Wout, bout, Wr, br,
           Wi1, bi1, Wi2, bi2, Wo1, bo1, Wo2, bo2, ng, nb, We1, be1, eg, eb,
           We2, be2):
    idx = node_features.reshape(-1).astype(jnp.int32)
    src = edge_index[0].astype(jnp.int32)
    dst = edge_index[1].astype(jnp.int32)
    loop = jnp.arange(N_TOTAL, dtype=jnp.int32)
    src_ext = jnp.concatenate([src, loop])
    dst_ext = jnp.concatenate([dst, loop])

    ones = jnp.ones(src_ext.shape[0], F32)
    deg_in = jax.ops.segment_sum(ones, dst_ext, num_segments=N_TOTAL)
    deg_out = jax.ops.segment_sum(ones, src_ext, num_segments=N_TOTAL)
    dis_in = lax.rsqrt(deg_in)
    dis_out = lax.rsqrt(deg_out)
    val_in = 0.5 * dis_in[src_ext] * dis_in[dst_ext]
    val_out = 0.5 * dis_out[src_ext] * dis_out[dst_ext]
    flat_in = dst_ext * jnp.int32(D) + idx[src_ext]
    flat_out = src_ext * jnp.int32(D) + idx[dst_ext]
    hist_in = jax.ops.segment_sum(val_in, flat_in,
                                  num_segments=N_TOTAL * D).reshape(N_TOTAL, D)
    hist_out = jax.ops.segment_sum(val_out, flat_out,
                                   num_segments=N_TOTAL * D).reshape(N_TOTAL, D)

    BD1 = jnp.zeros((D, D), F32).at[:H, :H].set(Wi1).at[H:, H:].set(Wo1)
    BD2 = jnp.zeros((D, D), F32).at[:H, :H].set(Wi2).at[H:, H:].set(Wo2)
    b1 = jnp.concatenate([bi1, bo1]).reshape(1, D)
    b2 = jnp.concatenate([bi2, bo2]).reshape(1, D)
    bias_comb = (0.5 * bin_ + 0.5 * bout + br).reshape(1, D)

    dots, exitv = _dense_pipeline(
        idx.reshape(B, 1, NUM_NODES), hist_in, hist_out, emb, Win, Wout, Wr,
        BD1, b1, BD2, b2, bias_comb, ng.reshape(1, D), nb.reshape(1, D),
        We1, be1.reshape(1, D), eg.reshape(1, D), eb.reshape(1, D),
        We2.reshape(1, D), be2.reshape(1, 1))

    edge_actions = dots.reshape(B, NUM_NODES * NUM_NODES)
    return jnp.concatenate([edge_actions, exitv], axis=-1)


# speedup vs baseline: 39.2209x; 39.2209x over previous
import functools

import jax
import jax.numpy as jnp
from jax import lax
from jax.experimental import pallas as pl
from jax.experimental.pallas import tpu as pltpu
from jax.experimental.pallas import tpu_sc as plsc

N_TOTAL = 32768
B = 256
NUM_NODES = 128
D = 128
H = 64
F32 = jnp.float32
I32 = jnp.int32

E = 524288
E_EXT = E + N_TOTAL
Q = 4096
ACC = Q * D

_INTERPRET = False


_DEG_PER_TILE = E_EXT // 32
_DEG_CHUNK = 2176
_DEG_NCHUNK = _DEG_PER_TILE // _DEG_CHUNK


def _deg_body(src_hbm, dst_hbm, out_hbm, sbuf, dbuf, ones, zbuf, tbuf,
              acc_in, acc_out, sem):
    c = lax.axis_index("c")
    s = lax.axis_index("s")

    def fill_ones(i, carry):
        ones[pl.ds(i * 16, 16)] = jnp.full((16,), 1.0, F32)
        return carry

    lax.fori_loop(0, 8, fill_ones, 0)

    def fill_zero(i, carry):
        zbuf[pl.ds(i * 16, 16)] = jnp.zeros((16,), F32)
        return carry

    lax.fori_loop(0, 128, fill_zero, 0)

    pltpu.sync_copy(zbuf, acc_in.at[pl.ds(s * 2048, 2048)])
    pltpu.sync_copy(zbuf, acc_out.at[pl.ds(s * 2048, 2048)])
    plsc.subcore_barrier()

    tile_base = (c * 16 + s) * (_DEG_PER_TILE // 128)

    def chunk(ch, carry):
        off = tile_base + ch * 8
        pltpu.sync_copy(src_hbm.at[pl.ds(off, 8)], sbuf)
        pltpu.sync_copy(dst_hbm.at[pl.ds(off, 8)], dbuf)
        cps = []
        for j in range(8):
            cps.append(pltpu.async_copy(ones, acc_in.at[dbuf.at[j]], sem,
                                        add=True))
            cps.append(pltpu.async_copy(ones, acc_out.at[sbuf.at[j]], sem,
                                        add=True))
        for cp in cps:
            cp.wait()
        return carry

    lax.fori_loop(0, 17, chunk, 0)
    plsc.subcore_barrier()

    pltpu.sync_copy(acc_in.at[pl.ds(s * 2048, 2048)], tbuf)
    pltpu.sync_copy(tbuf, out_hbm.at[pl.ds(c * 65536 + s * 2048, 2048)])
    pltpu.sync_copy(acc_out.at[pl.ds(s * 2048, 2048)], tbuf)
    pltpu.sync_copy(tbuf, out_hbm.at[pl.ds(c * 65536 + 32768 + s * 2048, 2048)])


def _sc_degrees(src_flat, dst_flat):
    src2 = src_flat.reshape(E_EXT // 128, 128)
    dst2 = dst_flat.reshape(E_EXT // 128, 128)
    mesh = plsc.VectorSubcoreMesh(core_axis_name="c", subcore_axis_name="s")
    f = pl.kernel(
        _deg_body,
        out_type=jax.ShapeDtypeStruct((131072,), F32),
        mesh=mesh,
        scratch_types=[
            pltpu.VMEM((8, 128), I32),
            pltpu.VMEM((8, 128), I32),
            pltpu.VMEM((128,), F32),
            pltpu.VMEM((2048,), F32),
            pltpu.VMEM((2048,), F32),
            pltpu.VMEM_SHARED((32768,), F32),
            pltpu.VMEM_SHARED((32768,), F32),
            pltpu.SemaphoreType.DMA,
        ],
    )
    return f(src2, dst2)


def _rsqrt_body(deg_ref, out_ref):
    out_ref[...] = lax.rsqrt(deg_ref[0] + deg_ref[1])


def _tc_rsqrt(degp):
    return pl.pallas_call(
        _rsqrt_body,
        out_shape=jax.ShapeDtypeStruct((2, 32768), F32),
        interpret=_INTERPRET,
    )(degp.reshape(2, 2, 32768))


_HIST_PER_TILE = E_EXT // 16
_HIST_CHUNK = 2176
_HIST_NCHUNK = _HIST_PER_TILE // _HIST_CHUNK


def _hist_body(src_hbm, dst_hbm, idx_hbm, dis2_hbm, hin_hbm, hout_hbm,
               idx_tab, dis_tab, ebuf_s, ebuf_d, fbuf, vbuf, zbuf, bounce,
               acc, sem):
    c = lax.axis_index("c")
    s = lax.axis_index("s")

    pltpu.sync_copy(idx_hbm, idx_tab)

    def fill_zero(i, carry):
        zbuf[pl.ds(i * 16, 16)] = jnp.zeros((16,), F32)
        return carry

    lax.fori_loop(0, 512, fill_zero, 0)

    lane = lax.broadcasted_iota(I32, (16,), 0)

    for p in range(8):
        direction = p // 4
        e8 = p % 4
        if e8 == 0:
            pltpu.sync_copy(dis2_hbm.at[pl.ds(direction * 32768, 32768)],
                            dis_tab)

        for bb in range(4):
            pltpu.sync_copy(zbuf, acc.at[pl.ds(s * 32768 + bb * 8192, 8192)])
        plsc.subcore_barrier()

        rng = e8 * 2 + c
        qbase = rng * Q

        def chunk(ch, carry):
            off = s * _HIST_PER_TILE + ch * _HIST_CHUNK
            pltpu.sync_copy(src_hbm.at[pl.ds(off, _HIST_CHUNK)], ebuf_s)
            pltpu.sync_copy(dst_hbm.at[pl.ds(off, _HIST_CHUNK)], ebuf_d)

            def group(i, carry2):
                goff = i * 16
                s16 = ebuf_s[pl.ds(goff, 16)]
                d16 = ebuf_d[pl.ds(goff, 16)]
                a = plsc.load_gather(dis_tab, [s16])
                b2 = plsc.load_gather(dis_tab, [d16])
                val = a * b2 * 0.5
                if direction == 0:
                    rows, cfrom = d16, s16
                else:
                    rows, cfrom = s16, d16
                cls = plsc.load_gather(idx_tab, [cfrom])
                loc = rows - qbase
                ok = (loc >= 0) & (loc < Q)
                flat = (loc * 128 + cls) & (ACC - 1)
                valm = jnp.where(ok, val, jnp.zeros((16,), F32))
                jv = jnp.full((16,), 0, I32) + (i >> 3)
                cv = lane + (i & 7) * 16
                plsc.store_scatter(fbuf, [jv, cv], flat)
                plsc.store_scatter(vbuf, [jv, cv], valm)
                return carry2

            lax.fori_loop(0, _HIST_CHUNK // 16, group, 0)

            cps = []
            for j in range(17):
                cps.append(pltpu.async_copy(vbuf.at[j], acc.at[fbuf.at[j]],
                                            sem, add=True))
            for cp in cps:
                cp.wait()
            return carry

        lax.fori_loop(0, _HIST_NCHUNK, chunk, 0)
        plsc.subcore_barrier()

        out_ref = hin_hbm if direction == 0 else hout_hbm
        for bb in range(4):
            pltpu.sync_copy(acc.at[pl.ds(s * 32768 + bb * 8192, 8192)], bounce)
            pltpu.sync_copy(bounce,
                            out_ref.at[pl.ds(rng * ACC + s * 32768
                                             + bb * 8192, 8192)])
        plsc.subcore_barrier()


def _sc_hist(src_flat, dst_flat, idx, dis2):
    mesh = plsc.VectorSubcoreMesh(core_axis_name="c", subcore_axis_name="s")
    f = pl.kernel(
        _hist_body,
        out_type=[jax.ShapeDtypeStruct((N_TOTAL * D,), F32),
                  jax.ShapeDtypeStruct((N_TOTAL * D,), F32)],
        mesh=mesh,
        scratch_types=[
            pltpu.VMEM((N_TOTAL,), I32),
            pltpu.VMEM((N_TOTAL,), F32),
            pltpu.VMEM((_HIST_CHUNK,), I32),
            pltpu.VMEM((_HIST_CHUNK,), I32),
            pltpu.VMEM((17, 128), I32),
            pltpu.VMEM((17, 128), F32),
            pltpu.VMEM((8192,), F32),
            pltpu.VMEM((8192,), F32),
            pltpu.VMEM_SHARED((ACC,), F32),
            pltpu.SemaphoreType.DMA,
        ],
        compiler_params=pltpu.CompilerParams(needs_layout_passes=False),
    )
    return f(src_flat, dst_flat, idx, dis2)


def _dense_body(idx_ref, hist_in_ref, hist_out_ref, emb_ref, Win_ref, Wout_ref,
                Wr_ref, BD1_ref, b1_ref, BD2_ref, b2_ref, bias_ref, ng_ref,
                nb_ref, We1_ref, be1_ref, eg_ref, eb_ref, We2r_ref, be2_ref,
                dots_ref, exit_ref, S_scr):
    b = pl.program_id(0)
    hi = lax.Precision.HIGHEST

    @pl.when(b == 0)
    def _():
        e = emb_ref[...]
        S_scr[0:D] = jnp.dot(e, Win_ref[...], preferred_element_type=F32,
                             precision=hi)
        S_scr[D:2 * D] = jnp.dot(e, Wout_ref[...], preferred_element_type=F32,
                                 precision=hi)
        S_scr[2 * D:3 * D] = jnp.dot(e, Wr_ref[...],
                                     preferred_element_type=F32, precision=hi)

    idxv = idx_ref[0]
    k_iota = lax.broadcasted_iota(I32, (D, NUM_NODES), 0)
    onehotT = (k_iota == idxv).astype(F32)

    xn = (jnp.dot(hist_in_ref[...], S_scr[0:D], preferred_element_type=F32,
                  precision=hi)
          + jnp.dot(hist_out_ref[...], S_scr[D:2 * D],
                    preferred_element_type=F32, precision=hi)
          + lax.dot_general(onehotT, S_scr[2 * D:3 * D],
                            (((0,), (0,)), ((), ())),
                            preferred_element_type=F32, precision=hi)
          + bias_ref[...])

    h1 = jnp.maximum(jnp.dot(xn, BD1_ref[...], preferred_element_type=F32,
                             precision=hi) + b1_ref[...], 0.0)
    x = jnp.dot(h1, BD2_ref[...], preferred_element_type=F32,
                precision=hi) + b2_ref[...]

    mu = jnp.mean(x, axis=1, keepdims=True)
    xc = x - mu
    var = jnp.mean(xc * xc, axis=1, keepdims=True)
    xln = xc * lax.rsqrt(var + 1e-5) * ng_ref[...] + nb_ref[...]

    m = jnp.mean(xln, axis=0, keepdims=True)
    hm = jnp.maximum(jnp.dot(m, We1_ref[...], preferred_element_type=F32,
                             precision=hi) + be1_ref[...], 0.0)
    mu2 = jnp.mean(hm, axis=1, keepdims=True)
    hc = hm - mu2
    var2 = jnp.mean(hc * hc, axis=1, keepdims=True)
    hln = hc * lax.rsqrt(var2 + 1e-5) * eg_ref[...] + eb_ref[...]
    exit_ref[pl.ds(b, 1), :] = (jnp.sum(hln * We2r_ref[...], axis=1,
                                        keepdims=True) + be2_ref[...])

    a = xln[:, :H]
    c2 = xln[:, H:]
    dots_ref[...] = lax.dot_general(
        a, c2, (((1,), (1,)), ((), ())), preferred_element_type=F32,
        precision=hi) * (1.0 / 8.0)


def _dense_pipeline(idx, hist_in, hist_out, emb, Win, Wout, Wr, BD1, b1, BD2,
                    b2, bias_comb, ng, nb, We1, be1, eg, eb, We2r, be2):
    full = lambda s: pl.BlockSpec(s, lambda b: tuple(0 for _ in s))
    dots, exitv = pl.pallas_call(
        _dense_body,
        grid=(B,),
        in_specs=[
            pl.BlockSpec((1, 1, NUM_NODES), lambda b: (b, 0, 0)),
            pl.BlockSpec((NUM_NODES, D), lambda b: (b, 0)),
            pl.BlockSpec((NUM_NODES, D), lambda b: (b, 0)),
            full((D, D)), full((D, D)), full((D, D)), full((D, D)),
            full((D, D)), full((1, D)), full((D, D)), full((1, D)),
            full((1, D)), full((1, D)), full((1, D)),
            full((D, D)), full((1, D)), full((1, D)), full((1, D)),
            full((1, D)), full((1, 1)),
        ],
        out_specs=[
            pl.BlockSpec((NUM_NODES, D), lambda b: (b, 0)),
            pl.BlockSpec((B, 1), lambda b: (0, 0)),
        ],
        out_shape=[
            jax.ShapeDtypeStruct((N_TOTAL, D), F32),
            jax.ShapeDtypeStruct((B, 1), F32),
        ],
        scratch_shapes=[pltpu.VMEM((3 * D, D), F32)],
        interpret=_INTERPRET,
    )(idx, hist_in, hist_out, emb, Win, Wout, Wr, BD1, b1, BD2, b2, bias_comb,
      ng, nb, We1, be1, eg, eb, We2r, be2)
    return dots, exitv


def kernel(node_features, edge_index, ptr, emb, Win, bin_, Wout, bout, Wr, br,
           Wi1, bi1, Wi2, bi2, Wo1, bo1, Wo2, bo2, ng, nb, We1, be1, eg, eb,
           We2, be2):
    idx = node_features.reshape(-1).astype(I32)
    src = edge_index[0].astype(I32)
    dst = edge_index[1].astype(I32)
    loop = jnp.arange(N_TOTAL, dtype=I32)
    src_ext = jnp.concatenate([src, loop])
    dst_ext = jnp.concatenate([dst, loop])

    degp = _sc_degrees(src_ext, dst_ext)
    dis2 = _tc_rsqrt(degp).reshape(-1)
    hin, hout = _sc_hist(src_ext, dst_ext, idx, dis2)
    hist_in = hin.reshape(N_TOTAL, D)
    hist_out = hout.reshape(N_TOTAL, D)

    BD1 = jnp.zeros((D, D), F32).at[:H, :H].set(Wi1).at[H:, H:].set(Wo1)
    BD2 = jnp.zeros((D, D), F32).at[:H, :H].set(Wi2).at[H:, H:].set(Wo2)
    b1 = jnp.concatenate([bi1, bo1]).reshape(1, D)
    b2 = jnp.concatenate([bi2, bo2]).reshape(1, D)
    bias_comb = (0.5 * bin_ + 0.5 * bout + br).reshape(1, D)

    dots, exitv = _dense_pipeline(
        idx.reshape(B, 1, NUM_NODES), hist_in, hist_out, emb, Win, Wout, Wr,
        BD1, b1, BD2, b2, bias_comb, ng.reshape(1, D), nb.reshape(1, D),
        We1, be1.reshape(1, D), eg.reshape(1, D), eb.reshape(1, D),
        We2.reshape(1, D), be2.reshape(1, 1))

    edge_actions = dots.reshape(B, NUM_NODES * NUM_NODES)
    return jnp.concatenate([edge_actions, exitv], axis=-1)


# speedup vs baseline: 56.3392x; 1.4365x over previous
import functools

import jax
import jax.numpy as jnp
from jax import lax
from jax.experimental import pallas as pl
from jax.experimental.pallas import tpu as pltpu
from jax.experimental.pallas import tpu_sc as plsc

N_TOTAL = 32768
B = 256
NUM_NODES = 128
D = 128
H = 64
F32 = jnp.float32
I32 = jnp.int32

E = 524288
E_EXT = E + N_TOTAL
Q = 4096
ACC = Q * D

_INTERPRET = False


_DEG_PER_TILE = E_EXT // 32
_DEG_CHUNK = 2176
_DEG_NCHUNK = _DEG_PER_TILE // _DEG_CHUNK


def _deg_body(src_hbm, dst_hbm, out_hbm, sbuf, dbuf, ones, zbuf, tbuf,
              acc_in, acc_out, sem):
    c = lax.axis_index("c")
    s = lax.axis_index("s")

    def fill_ones(i, carry):
        ones[pl.ds(i * 16, 16)] = jnp.full((16,), 1.0, F32)
        return carry

    lax.fori_loop(0, 8, fill_ones, 0)

    def fill_zero(i, carry):
        zbuf[pl.ds(i * 16, 16)] = jnp.zeros((16,), F32)
        return carry

    lax.fori_loop(0, 128, fill_zero, 0)

    pltpu.sync_copy(zbuf, acc_in.at[pl.ds(s * 2048, 2048)])
    pltpu.sync_copy(zbuf, acc_out.at[pl.ds(s * 2048, 2048)])
    plsc.subcore_barrier()

    tile_base = (c * 16 + s) * (_DEG_PER_TILE // 128)

    def chunk(ch, carry):
        off = tile_base + ch * 8
        pltpu.sync_copy(src_hbm.at[pl.ds(off, 8)], sbuf)
        pltpu.sync_copy(dst_hbm.at[pl.ds(off, 8)], dbuf)
        cps = []
        for j in range(8):
            cps.append(pltpu.async_copy(ones, acc_in.at[dbuf.at[j]], sem,
                                        add=True))
            cps.append(pltpu.async_copy(ones, acc_out.at[sbuf.at[j]], sem,
                                        add=True))
        for cp in cps:
            cp.wait()
        return carry

    lax.fori_loop(0, 17, chunk, 0)
    plsc.subcore_barrier()

    pltpu.sync_copy(acc_in.at[pl.ds(s * 2048, 2048)], tbuf)
    pltpu.sync_copy(tbuf, out_hbm.at[pl.ds(c * 65536 + s * 2048, 2048)])
    pltpu.sync_copy(acc_out.at[pl.ds(s * 2048, 2048)], tbuf)
    pltpu.sync_copy(tbuf, out_hbm.at[pl.ds(c * 65536 + 32768 + s * 2048, 2048)])


def _sc_degrees(src_flat, dst_flat):
    src2 = src_flat.reshape(E_EXT // 128, 128)
    dst2 = dst_flat.reshape(E_EXT // 128, 128)
    mesh = plsc.VectorSubcoreMesh(core_axis_name="c", subcore_axis_name="s")
    f = pl.kernel(
        _deg_body,
        out_type=jax.ShapeDtypeStruct((131072,), F32),
        mesh=mesh,
        scratch_types=[
            pltpu.VMEM((8, 128), I32),
            pltpu.VMEM((8, 128), I32),
            pltpu.VMEM((128,), F32),
            pltpu.VMEM((2048,), F32),
            pltpu.VMEM((2048,), F32),
            pltpu.VMEM_SHARED((32768,), F32),
            pltpu.VMEM_SHARED((32768,), F32),
            pltpu.SemaphoreType.DMA,
        ],
    )
    return f(src2, dst2)


def _rsqrt_body(deg_ref, out_ref):
    out_ref[...] = lax.rsqrt(deg_ref[0] + deg_ref[1])


def _tc_rsqrt(degp):
    return pl.pallas_call(
        _rsqrt_body,
        out_shape=jax.ShapeDtypeStruct((2, 32768), F32),
        interpret=_INTERPRET,
    )(degp.reshape(2, 2, 32768))


_HIST_PER_TILE = E_EXT // 16
_HIST_CHUNK = 2176
_HIST_NCHUNK = _HIST_PER_TILE // _HIST_CHUNK


def _hist_body(src_hbm, dst_hbm, idx_hbm, dis2_hbm, hin_hbm, hout_hbm,
               idx_tab, dis_tab, ebuf_s, ebuf_d, fbuf, vbuf, zbuf, bounce,
               acc, sem):
    c = lax.axis_index("c")
    s = lax.axis_index("s")

    pltpu.sync_copy(idx_hbm, idx_tab)

    def fill_zero(i, carry):
        zbuf[pl.ds(i * 16, 16)] = jnp.zeros((16,), F32)
        return carry

    lax.fori_loop(0, 512, fill_zero, 0)

    lane = lax.broadcasted_iota(I32, (16,), 0)

    for p in range(8):
        direction = p // 4
        e8 = p % 4
        if e8 == 0:
            pltpu.sync_copy(dis2_hbm.at[pl.ds(direction * 32768, 32768)],
                            dis_tab)

        for bb in range(4):
            pltpu.sync_copy(zbuf, acc.at[pl.ds(s * 32768 + bb * 8192, 8192)])
        plsc.subcore_barrier()

        rng = e8 * 2 + c
        qbase = rng * Q

        def chunk(ch, carry):
            off = s * _HIST_PER_TILE + ch * _HIST_CHUNK
            pltpu.sync_copy(src_hbm.at[pl.ds(off, _HIST_CHUNK)], ebuf_s)
            pltpu.sync_copy(dst_hbm.at[pl.ds(off, _HIST_CHUNK)], ebuf_d)

            def group(i, carry2):
                goff = i * 16
                s16 = ebuf_s[pl.ds(goff, 16)]
                d16 = ebuf_d[pl.ds(goff, 16)]
                a = plsc.load_gather(dis_tab, [s16])
                b2 = plsc.load_gather(dis_tab, [d16])
                val = a * b2 * 0.5
                if direction == 0:
                    rows, cfrom = d16, s16
                else:
                    rows, cfrom = s16, d16
                cls = plsc.load_gather(idx_tab, [cfrom])
                loc = rows - qbase
                ok = (loc >= 0) & (loc < Q)
                flat = (loc * 128 + cls) & (ACC - 1)
                valm = jnp.where(ok, val, jnp.zeros((16,), F32))
                jv = jnp.full((16,), 0, I32) + (i >> 3)
                cv = lane + (i & 7) * 16
                plsc.store_scatter(fbuf, [jv, cv], flat)
                plsc.store_scatter(vbuf, [jv, cv], valm)
                return carry2

            lax.fori_loop(0, _HIST_CHUNK // 16, group, 0)

            cps = []
            for j in range(17):
                cps.append(pltpu.async_copy(vbuf.at[j], acc.at[fbuf.at[j]],
                                            sem, add=True))
            for cp in cps:
                cp.wait()
            return carry

        lax.fori_loop(0, _HIST_NCHUNK, chunk, 0)
        plsc.subcore_barrier()

        out_ref = hin_hbm if direction == 0 else hout_hbm
        for bb in range(4):
            pltpu.sync_copy(acc.at[pl.ds(s * 32768 + bb * 8192, 8192)], bounce)
            pltpu.sync_copy(bounce,
                            out_ref.at[pl.ds(rng * ACC + s * 32768
                                             + bb * 8192, 8192)])
        plsc.subcore_barrier()


def _sc_hist(src_flat, dst_flat, idx, dis2):
    mesh = plsc.VectorSubcoreMesh(core_axis_name="c", subcore_axis_name="s")
    f = pl.kernel(
        _hist_body,
        out_type=[jax.ShapeDtypeStruct((N_TOTAL * D,), F32),
                  jax.ShapeDtypeStruct((N_TOTAL * D,), F32)],
        mesh=mesh,
        scratch_types=[
            pltpu.VMEM((N_TOTAL,), I32),
            pltpu.VMEM((N_TOTAL,), F32),
            pltpu.VMEM((_HIST_CHUNK,), I32),
            pltpu.VMEM((_HIST_CHUNK,), I32),
            pltpu.VMEM((17, 128), I32),
            pltpu.VMEM((17, 128), F32),
            pltpu.VMEM((8192,), F32),
            pltpu.VMEM((8192,), F32),
            pltpu.VMEM_SHARED((ACC,), F32),
            pltpu.SemaphoreType.DMA,
        ],
        compiler_params=pltpu.CompilerParams(needs_layout_passes=False),
    )
    return f(src_flat, dst_flat, idx, dis2)


_GB = 4
_RB = _GB * NUM_NODES


def _dense_body(idx_ref, hist_in_ref, hist_out_ref, emb_ref, Win_ref, Wout_ref,
                Wr_ref, BD1_ref, b1_ref, BD2_ref, b2_ref, bias_ref, ng_ref,
                nb_ref, We1_ref, be1_ref, eg_ref, eb_ref, We2r_ref, be2_ref,
                dots_ref, exit_ref, S_scr):
    b = pl.program_id(0)
    hi = lax.Precision.HIGHEST

    @pl.when(b == 0)
    def _():
        e = emb_ref[...]
        S_scr[0:D] = jnp.dot(e, Win_ref[...], preferred_element_type=F32,
                             precision=hi)
        S_scr[D:2 * D] = jnp.dot(e, Wout_ref[...], preferred_element_type=F32,
                                 precision=hi)
        S_scr[2 * D:3 * D] = jnp.dot(e, Wr_ref[...],
                                     preferred_element_type=F32, precision=hi)

    hp = lax.Precision.DEFAULT
    idxv = idx_ref[0]
    k_iota = lax.broadcasted_iota(I32, (D, _RB), 0)
    onehotT = (k_iota == idxv).astype(F32)

    xn = (jnp.dot(hist_in_ref[...], S_scr[0:D], preferred_element_type=F32,
                  precision=hp)
          + jnp.dot(hist_out_ref[...], S_scr[D:2 * D],
                    preferred_element_type=F32, precision=hp)
          + lax.dot_general(onehotT, S_scr[2 * D:3 * D],
                            (((0,), (0,)), ((), ())),
                            preferred_element_type=F32, precision=hp)
          + bias_ref[...])

    h1 = jnp.maximum(jnp.dot(xn, BD1_ref[...], preferred_element_type=F32,
                             precision=hp) + b1_ref[...], 0.0)
    x = jnp.dot(h1, BD2_ref[...], preferred_element_type=F32,
                precision=hp) + b2_ref[...]

    mu = jnp.mean(x, axis=1, keepdims=True)
    xc = x - mu
    var = jnp.mean(xc * xc, axis=1, keepdims=True)
    xln = xc * lax.rsqrt(var + 1e-5) * ng_ref[...] + nb_ref[...]

    m = jnp.concatenate(
        [jnp.mean(xln[g * NUM_NODES:(g + 1) * NUM_NODES], axis=0,
                  keepdims=True) for g in range(_GB)], axis=0)
    hm = jnp.maximum(jnp.dot(m, We1_ref[...], preferred_element_type=F32,
                             precision=hi) + be1_ref[...], 0.0)
    mu2 = jnp.mean(hm, axis=1, keepdims=True)
    hc = hm - mu2
    var2 = jnp.mean(hc * hc, axis=1, keepdims=True)
    hln = hc * lax.rsqrt(var2 + 1e-5) * eg_ref[...] + eb_ref[...]
    exit_ref[pl.ds(b * _GB, _GB), :] = (
        jnp.sum(hln * We2r_ref[...], axis=1, keepdims=True) + be2_ref[...])

    for g in range(_GB):
        blk = xln[g * NUM_NODES:(g + 1) * NUM_NODES]
        dots_ref[g * NUM_NODES:(g + 1) * NUM_NODES, :] = lax.dot_general(
            blk[:, :H], blk[:, H:], (((1,), (1,)), ((), ())),
            preferred_element_type=F32, precision=hp) * (1.0 / 8.0)


def _dense_pipeline(idx, hist_in, hist_out, emb, Win, Wout, Wr, BD1, b1, BD2,
                    b2, bias_comb, ng, nb, We1, be1, eg, eb, We2r, be2):
    full = lambda s: pl.BlockSpec(s, lambda b: tuple(0 for _ in s))
    dots, exitv = pl.pallas_call(
        _dense_body,
        grid=(B // _GB,),
        in_specs=[
            pl.BlockSpec((1, 1, _RB), lambda b: (b, 0, 0)),
            pl.BlockSpec((_RB, D), lambda b: (b, 0)),
            pl.BlockSpec((_RB, D), lambda b: (b, 0)),
            full((D, D)), full((D, D)), full((D, D)), full((D, D)),
            full((D, D)), full((1, D)), full((D, D)), full((1, D)),
            full((1, D)), full((1, D)), full((1, D)),
            full((D, D)), full((1, D)), full((1, D)), full((1, D)),
            full((1, D)), full((1, 1)),
        ],
        out_specs=[
            pl.BlockSpec((_RB, D), lambda b: (b, 0)),
            pl.BlockSpec((B, 1), lambda b: (0, 0)),
        ],
        out_shape=[
            jax.ShapeDtypeStruct((N_TOTAL, D), F32),
            jax.ShapeDtypeStruct((B, 1), F32),
        ],
        scratch_shapes=[pltpu.VMEM((3 * D, D), F32)],
        interpret=_INTERPRET,
    )(idx, hist_in, hist_out, emb, Win, Wout, Wr, BD1, b1, BD2, b2, bias_comb,
      ng, nb, We1, be1, eg, eb, We2r, be2)
    return dots, exitv


def kernel(node_features, edge_index, ptr, emb, Win, bin_, Wout, bout, Wr, br,
           Wi1, bi1, Wi2, bi2, Wo1, bo1, Wo2, bo2, ng, nb, We1, be1, eg, eb,
           We2, be2):
    idx = node_features.reshape(-1).astype(I32)
    src = edge_index[0].astype(I32)
    dst = edge_index[1].astype(I32)
    loop = jnp.arange(N_TOTAL, dtype=I32)
    src_ext = jnp.concatenate([src, loop])
    dst_ext = jnp.concatenate([dst, loop])

    degp = _sc_degrees(src_ext, dst_ext)
    dis2 = _tc_rsqrt(degp).reshape(-1)
    hin, hout = _sc_hist(src_ext, dst_ext, idx, dis2)
    hist_in = hin.reshape(N_TOTAL, D)
    hist_out = hout.reshape(N_TOTAL, D)

    BD1 = jnp.zeros((D, D), F32).at[:H, :H].set(Wi1).at[H:, H:].set(Wo1)
    BD2 = jnp.zeros((D, D), F32).at[:H, :H].set(Wi2).at[H:, H:].set(Wo2)
    b1 = jnp.concatenate([bi1, bo1]).reshape(1, D)
    b2 = jnp.concatenate([bi2, bo2]).reshape(1, D)
    bias_comb = (0.5 * bin_ + 0.5 * bout + br).reshape(1, D)

    dots, exitv = _dense_pipeline(
        idx.reshape(B // _GB, 1, _RB), hist_in, hist_out, emb, Win, Wout, Wr,
        BD1, b1, BD2, b2, bias_comb, ng.reshape(1, D), nb.reshape(1, D),
        We1, be1.reshape(1, D), eg.reshape(1, D), eb.reshape(1, D),
        We2.reshape(1, D), be2.reshape(1, 1))

    edge_actions = dots.reshape(B, NUM_NODES * NUM_NODES)
    return jnp.concatenate([edge_actions, exitv], axis=-1)


# speedup vs baseline: 58.6454x; 1.0409x over previous
import functools

import jax
import jax.numpy as jnp
from jax import lax
from jax.experimental import pallas as pl
from jax.experimental.pallas import tpu as pltpu
from jax.experimental.pallas import tpu_sc as plsc

N_TOTAL = 32768
B = 256
NUM_NODES = 128
D = 128
H = 64
F32 = jnp.float32
I32 = jnp.int32

E = 524288
E_EXT = E + N_TOTAL
Q = 4096
ACC = Q * D

_INTERPRET = False


_DEG_PER_TILE = E_EXT // 32
_DEG_CHUNK = 2176
_DEG_NCHUNK = _DEG_PER_TILE // _DEG_CHUNK


def _deg_body(src_hbm, dst_hbm, out_hbm, sbuf, dbuf, ones, zbuf, tbuf,
              acc_in, acc_out, sem):
    c = lax.axis_index("c")
    s = lax.axis_index("s")

    def fill_ones(i, carry):
        ones[pl.ds(i * 16, 16)] = jnp.full((16,), 1.0, F32)
        return carry

    lax.fori_loop(0, 8, fill_ones, 0)

    def fill_zero(i, carry):
        zbuf[pl.ds(i * 16, 16)] = jnp.zeros((16,), F32)
        return carry

    lax.fori_loop(0, 128, fill_zero, 0)

    pltpu.sync_copy(zbuf, acc_in.at[pl.ds(s * 2048, 2048)])
    pltpu.sync_copy(zbuf, acc_out.at[pl.ds(s * 2048, 2048)])
    plsc.subcore_barrier()

    tile_base = (c * 16 + s) * (_DEG_PER_TILE // 128)

    def chunk(ch, carry):
        off = tile_base + ch * 8
        pltpu.sync_copy(src_hbm.at[pl.ds(off, 8)], sbuf)
        pltpu.sync_copy(dst_hbm.at[pl.ds(off, 8)], dbuf)
        cps = []
        for j in range(8):
            cps.append(pltpu.async_copy(ones, acc_in.at[dbuf.at[j]], sem,
                                        add=True))
            cps.append(pltpu.async_copy(ones, acc_out.at[sbuf.at[j]], sem,
                                        add=True))
        for cp in cps:
            cp.wait()
        return carry

    lax.fori_loop(0, 17, chunk, 0)
    plsc.subcore_barrier()

    pltpu.sync_copy(acc_in.at[pl.ds(s * 2048, 2048)], tbuf)
    pltpu.sync_copy(tbuf, out_hbm.at[pl.ds(c * 65536 + s * 2048, 2048)])
    pltpu.sync_copy(acc_out.at[pl.ds(s * 2048, 2048)], tbuf)
    pltpu.sync_copy(tbuf, out_hbm.at[pl.ds(c * 65536 + 32768 + s * 2048, 2048)])


def _sc_degrees(src_flat, dst_flat):
    src2 = src_flat.reshape(E_EXT // 128, 128)
    dst2 = dst_flat.reshape(E_EXT // 128, 128)
    mesh = plsc.VectorSubcoreMesh(core_axis_name="c", subcore_axis_name="s")
    f = pl.kernel(
        _deg_body,
        out_type=jax.ShapeDtypeStruct((131072,), F32),
        mesh=mesh,
        scratch_types=[
            pltpu.VMEM((8, 128), I32),
            pltpu.VMEM((8, 128), I32),
            pltpu.VMEM((128,), F32),
            pltpu.VMEM((2048,), F32),
            pltpu.VMEM((2048,), F32),
            pltpu.VMEM_SHARED((32768,), F32),
            pltpu.VMEM_SHARED((32768,), F32),
            pltpu.SemaphoreType.DMA,
        ],
    )
    return f(src2, dst2)


def _rsqrt_body(deg_ref, out_ref):
    out_ref[...] = lax.rsqrt(deg_ref[0] + deg_ref[1])


def _tc_rsqrt(degp):
    return pl.pallas_call(
        _rsqrt_body,
        out_shape=jax.ShapeDtypeStruct((2, 32768), F32),
        interpret=_INTERPRET,
    )(degp.reshape(2, 2, 32768))


_HIST_PER_TILE = E_EXT // 16
_HIST_CHUNK = 2176
_HIST_NCHUNK = _HIST_PER_TILE // _HIST_CHUNK


def _hist_body(src_hbm, dst_hbm, idx_hbm, dis2_hbm, hin_hbm, hout_hbm,
               idx_tab, dis_tab, ebuf_s, ebuf_d, fbuf_f, vbuf_f, fbuf2, vbuf2,
               zbuf, bounce, acc, sem):
    c = lax.axis_index("c")
    s = lax.axis_index("s")

    pltpu.sync_copy(idx_hbm, idx_tab)

    def fill_zero(i, carry):
        zbuf[pl.ds(i * 16, 16)] = jnp.zeros((16,), F32)
        return carry

    lax.fori_loop(0, 512, fill_zero, 0)

    lane = lax.broadcasted_iota(I32, (16,), 0)

    for p in range(8):
        direction = p // 4
        e8 = p % 4
        if e8 == 0:
            pltpu.sync_copy(dis2_hbm.at[pl.ds(direction * 32768, 32768)],
                            dis_tab)

        for bb in range(4):
            pltpu.sync_copy(zbuf, acc.at[pl.ds(s * 32768 + bb * 8192, 8192)])
        plsc.subcore_barrier()

        rng = e8 * 2 + c
        qbase = rng * Q

        def chunk(ch, carry):
            off = s * _HIST_PER_TILE + ch * _HIST_CHUNK
            pltpu.sync_copy(src_hbm.at[pl.ds(off, _HIST_CHUNK)], ebuf_s)
            pltpu.sync_copy(dst_hbm.at[pl.ds(off, _HIST_CHUNK)], ebuf_d)

            def group(i, cur):
                goff = i * 16
                s16 = ebuf_s[pl.ds(goff, 16)]
                d16 = ebuf_d[pl.ds(goff, 16)]
                a = plsc.load_gather(dis_tab, [s16])
                b2 = plsc.load_gather(dis_tab, [d16])
                val = a * b2 * 0.5
                if direction == 0:
                    rows, cfrom = d16, s16
                else:
                    rows, cfrom = s16, d16
                cls = plsc.load_gather(idx_tab, [cfrom])
                loc = rows - qbase
                ok = (loc >= 0) & (loc < Q)
                flat = (loc * 128 + cls) & (ACC - 1)
                plsc.store_compressed(fbuf_f.at[pl.ds(cur, 16)], flat, mask=ok)
                plsc.store_compressed(vbuf_f.at[pl.ds(cur, 16)], val, mask=ok)
                return cur + jnp.sum(ok.astype(I32))

            cursor = lax.fori_loop(0, _HIST_CHUNK // 16, group, jnp.int32(0))

            okall = lane < 16
            zval = jnp.zeros((16,), F32)

            def pad(k, carry2):
                po = cursor + k * 16
                plsc.store_compressed(fbuf_f.at[pl.ds(po, 16)],
                                      (lane + po) & (ACC - 1), mask=okall)
                plsc.store_compressed(vbuf_f.at[pl.ds(po, 16)], zval, mask=okall)
                return carry2

            lax.fori_loop(0, 8, pad, 0)
            nrows = (cursor + 127) >> 7

            def torow(k, carry2):
                fv = fbuf_f[pl.ds(k * 16, 16)]
                vv = vbuf_f[pl.ds(k * 16, 16)]
                jv = jnp.full((16,), 0, I32) + (k >> 3)
                cv = lane + (k & 7) * 16
                plsc.store_scatter(fbuf2, [jv, cv], fv)
                plsc.store_scatter(vbuf2, [jv, cv], vv)
                return carry2

            lax.fori_loop(0, nrows * 8, torow, 0)

            def fire(j, carry2):
                pltpu.sync_copy(vbuf2.at[j], acc.at[fbuf2.at[j]], add=True)
                return carry2

            lax.fori_loop(0, nrows, fire, 0)
            return carry

        lax.fori_loop(0, _HIST_NCHUNK, chunk, 0)
        plsc.subcore_barrier()

        out_ref = hin_hbm if direction == 0 else hout_hbm
        for bb in range(4):
            pltpu.sync_copy(acc.at[pl.ds(s * 32768 + bb * 8192, 8192)], bounce)
            pltpu.sync_copy(bounce,
                            out_ref.at[pl.ds(rng * ACC + s * 32768
                                             + bb * 8192, 8192)])
        plsc.subcore_barrier()


def _sc_hist(src_flat, dst_flat, idx, dis2):
    mesh = plsc.VectorSubcoreMesh(core_axis_name="c", subcore_axis_name="s")
    f = pl.kernel(
        _hist_body,
        out_type=[jax.ShapeDtypeStruct((N_TOTAL * D,), F32),
                  jax.ShapeDtypeStruct((N_TOTAL * D,), F32)],
        mesh=mesh,
        scratch_types=[
            pltpu.VMEM((N_TOTAL,), I32),
            pltpu.VMEM((N_TOTAL,), F32),
            pltpu.VMEM((_HIST_CHUNK,), I32),
            pltpu.VMEM((_HIST_CHUNK,), I32),
            pltpu.VMEM((_HIST_CHUNK + 128,), I32),
            pltpu.VMEM((_HIST_CHUNK + 128,), F32),
            pltpu.VMEM((18, 128), I32),
            pltpu.VMEM((18, 128), F32),
            pltpu.VMEM((8192,), F32),
            pltpu.VMEM((8192,), F32),
            pltpu.VMEM_SHARED((ACC,), F32),
            pltpu.SemaphoreType.DMA,
        ],
        compiler_params=pltpu.CompilerParams(needs_layout_passes=False),
    )
    return f(src_flat, dst_flat, idx, dis2)


_GB = 4
_RB = _GB * NUM_NODES


def _dense_body(idx_ref, hist_in_ref, hist_out_ref, emb_ref, Win_ref, Wout_ref,
                Wr_ref, BD1_ref, b1_ref, BD2_ref, b2_ref, bias_ref, ng_ref,
                nb_ref, We1_ref, be1_ref, eg_ref, eb_ref, We2r_ref, be2_ref,
                dots_ref, exit_ref, S_scr):
    b = pl.program_id(0)
    hi = lax.Precision.HIGHEST

    @pl.when(b == 0)
    def _():
        e = emb_ref[...]
        S_scr[0:D] = jnp.dot(e, Win_ref[...], preferred_element_type=F32,
                             precision=hi)
        S_scr[D:2 * D] = jnp.dot(e, Wout_ref[...], preferred_element_type=F32,
                                 precision=hi)
        S_scr[2 * D:3 * D] = jnp.dot(e, Wr_ref[...],
                                     preferred_element_type=F32, precision=hi)

    hp = lax.Precision.DEFAULT
    idxv = idx_ref[0]
    k_iota = lax.broadcasted_iota(I32, (D, _RB), 0)
    onehotT = (k_iota == idxv).astype(F32)

    xn = (jnp.dot(hist_in_ref[...], S_scr[0:D], preferred_element_type=F32,
                  precision=hp)
          + jnp.dot(hist_out_ref[...], S_scr[D:2 * D],
                    preferred_element_type=F32, precision=hp)
          + lax.dot_general(onehotT, S_scr[2 * D:3 * D],
                            (((0,), (0,)), ((), ())),
                            preferred_element_type=F32, precision=hp)
          + bias_ref[...])

    h1 = jnp.maximum(jnp.dot(xn, BD1_ref[...], preferred_element_type=F32,
                             precision=hp) + b1_ref[...], 0.0)
    x = jnp.dot(h1, BD2_ref[...], preferred_element_type=F32,
                precision=hp) + b2_ref[...]

    mu = jnp.mean(x, axis=1, keepdims=True)
    xc = x - mu
    var = jnp.mean(xc * xc, axis=1, keepdims=True)
    xln = xc * lax.rsqrt(var + 1e-5) * ng_ref[...] + nb_ref[...]

    m = jnp.concatenate(
        [jnp.mean(xln[g * NUM_NODES:(g + 1) * NUM_NODES], axis=0,
                  keepdims=True) for g in range(_GB)], axis=0)
    hm = jnp.maximum(jnp.dot(m, We1_ref[...], preferred_element_type=F32,
                             precision=hi) + be1_ref[...], 0.0)
    mu2 = jnp.mean(hm, axis=1, keepdims=True)
    hc = hm - mu2
    var2 = jnp.mean(hc * hc, axis=1, keepdims=True)
    hln = hc * lax.rsqrt(var2 + 1e-5) * eg_ref[...] + eb_ref[...]
    exit_ref[pl.ds(b * _GB, _GB), :] = (
        jnp.sum(hln * We2r_ref[...], axis=1, keepdims=True) + be2_ref[...])

    for g in range(_GB):
        blk = xln[g * NUM_NODES:(g + 1) * NUM_NODES]
        dots_ref[g * NUM_NODES:(g + 1) * NUM_NODES, :] = lax.dot_general(
            blk[:, :H], blk[:, H:], (((1,), (1,)), ((), ())),
            preferred_element_type=F32, precision=hp) * (1.0 / 8.0)


def _dense_pipeline(idx, hist_in, hist_out, emb, Win, Wout, Wr, BD1, b1, BD2,
                    b2, bias_comb, ng, nb, We1, be1, eg, eb, We2r, be2):
    full = lambda s: pl.BlockSpec(s, lambda b: tuple(0 for _ in s))
    dots, exitv = pl.pallas_call(
        _dense_body,
        grid=(B // _GB,),
        in_specs=[
            pl.BlockSpec((1, 1, _RB), lambda b: (b, 0, 0)),
            pl.BlockSpec((_RB, D), lambda b: (b, 0)),
            pl.BlockSpec((_RB, D), lambda b: (b, 0)),
            full((D, D)), full((D, D)), full((D, D)), full((D, D)),
            full((D, D)), full((1, D)), full((D, D)), full((1, D)),
            full((1, D)), full((1, D)), full((1, D)),
            full((D, D)), full((1, D)), full((1, D)), full((1, D)),
            full((1, D)), full((1, 1)),
        ],
        out_specs=[
            pl.BlockSpec((_RB, D), lambda b: (b, 0)),
            pl.BlockSpec((B, 1), lambda b: (0, 0)),
        ],
        out_shape=[
            jax.ShapeDtypeStruct((N_TOTAL, D), F32),
            jax.ShapeDtypeStruct((B, 1), F32),
        ],
        scratch_shapes=[pltpu.VMEM((3 * D, D), F32)],
        interpret=_INTERPRET,
    )(idx, hist_in, hist_out, emb, Win, Wout, Wr, BD1, b1, BD2, b2, bias_comb,
      ng, nb, We1, be1, eg, eb, We2r, be2)
    return dots, exitv


def kernel(node_features, edge_index, ptr, emb, Win, bin_, Wout, bout, Wr, br,
           Wi1, bi1, Wi2, bi2, Wo1, bo1, Wo2, bo2, ng, nb, We1, be1, eg, eb,
           We2, be2):
    idx = node_features.reshape(-1).astype(I32)
    src = edge_index[0].astype(I32)
    dst = edge_index[1].astype(I32)
    loop = jnp.arange(N_TOTAL, dtype=I32)
    src_ext = jnp.concatenate([src, loop])
    dst_ext = jnp.concatenate([dst, loop])

    degp = _sc_degrees(src_ext, dst_ext)
    dis2 = _tc_rsqrt(degp).reshape(-1)
    hin, hout = _sc_hist(src_ext, dst_ext, idx, dis2)
    hist_in = hin.reshape(N_TOTAL, D)
    hist_out = hout.reshape(N_TOTAL, D)

    BD1 = jnp.zeros((D, D), F32).at[:H, :H].set(Wi1).at[H:, H:].set(Wo1)
    BD2 = jnp.zeros((D, D), F32).at[:H, :H].set(Wi2).at[H:, H:].set(Wo2)
    b1 = jnp.concatenate([bi1, bo1]).reshape(1, D)
    b2 = jnp.concatenate([bi2, bo2]).reshape(1, D)
    bias_comb = (0.5 * bin_ + 0.5 * bout + br).reshape(1, D)

    dots, exitv = _dense_pipeline(
        idx.reshape(B // _GB, 1, _RB), hist_in, hist_out, emb, Win, Wout, Wr,
        BD1, b1, BD2, b2, bias_comb, ng.reshape(1, D), nb.reshape(1, D),
        We1, be1.reshape(1, D), eg.reshape(1, D), eb.reshape(1, D),
        We2.reshape(1, D), be2.reshape(1, 1))

    edge_actions = dots.reshape(B, NUM_NODES * NUM_NODES)
    return jnp.concatenate([edge_actions, exitv], axis=-1)


# speedup vs baseline: 59.4894x; 1.0144x over previous
import functools

import jax
import jax.numpy as jnp
from jax import lax
from jax.experimental import pallas as pl
from jax.experimental.pallas import tpu as pltpu
from jax.experimental.pallas import tpu_sc as plsc

N_TOTAL = 32768
B = 256
NUM_NODES = 128
D = 128
H = 64
F32 = jnp.float32
I32 = jnp.int32

E = 524288
E_EXT = E + N_TOTAL
Q = 4096
ACC = Q * D

_INTERPRET = False


_DEG_PER_TILE = E_EXT // 32
_DEG_CHUNK = 2176
_DEG_NCHUNK = _DEG_PER_TILE // _DEG_CHUNK


def _deg_body(src_hbm, dst_hbm, out_hbm, sbuf, dbuf, ones, zbuf, tbuf,
              acc_in, acc_out, sem):
    c = lax.axis_index("c")
    s = lax.axis_index("s")

    def fill_ones(i, carry):
        ones[pl.ds(i * 16, 16)] = jnp.full((16,), 1.0, F32)
        return carry

    lax.fori_loop(0, 8, fill_ones, 0)

    def fill_zero(i, carry):
        zbuf[pl.ds(i * 16, 16)] = jnp.zeros((16,), F32)
        return carry

    lax.fori_loop(0, 128, fill_zero, 0)

    pltpu.sync_copy(zbuf, acc_in.at[pl.ds(s * 2048, 2048)])
    pltpu.sync_copy(zbuf, acc_out.at[pl.ds(s * 2048, 2048)])
    plsc.subcore_barrier()

    tile_base = (c * 16 + s) * (_DEG_PER_TILE // 128)

    def chunk(ch, carry):
        off = tile_base + ch * 8
        pltpu.sync_copy(src_hbm.at[pl.ds(off, 8)], sbuf)
        pltpu.sync_copy(dst_hbm.at[pl.ds(off, 8)], dbuf)
        cps = []
        for j in range(8):
            cps.append(pltpu.async_copy(ones, acc_in.at[dbuf.at[j]], sem,
                                        add=True))
            cps.append(pltpu.async_copy(ones, acc_out.at[sbuf.at[j]], sem,
                                        add=True))
        for cp in cps:
            cp.wait()
        return carry

    lax.fori_loop(0, 17, chunk, 0)
    plsc.subcore_barrier()

    pltpu.sync_copy(acc_in.at[pl.ds(s * 2048, 2048)], tbuf)
    pltpu.sync_copy(tbuf, out_hbm.at[pl.ds(c * 65536 + s * 2048, 2048)])
    pltpu.sync_copy(acc_out.at[pl.ds(s * 2048, 2048)], tbuf)
    pltpu.sync_copy(tbuf, out_hbm.at[pl.ds(c * 65536 + 32768 + s * 2048, 2048)])


def _sc_degrees(src_flat, dst_flat):
    src2 = src_flat.reshape(E_EXT // 128, 128)
    dst2 = dst_flat.reshape(E_EXT // 128, 128)
    mesh = plsc.VectorSubcoreMesh(core_axis_name="c", subcore_axis_name="s")
    f = pl.kernel(
        _deg_body,
        out_type=jax.ShapeDtypeStruct((131072,), F32),
        mesh=mesh,
        scratch_types=[
            pltpu.VMEM((8, 128), I32),
            pltpu.VMEM((8, 128), I32),
            pltpu.VMEM((128,), F32),
            pltpu.VMEM((2048,), F32),
            pltpu.VMEM((2048,), F32),
            pltpu.VMEM_SHARED((32768,), F32),
            pltpu.VMEM_SHARED((32768,), F32),
            pltpu.SemaphoreType.DMA,
        ],
    )
    return f(src2, dst2)


def _rsqrt_body(deg_ref, out_ref):
    out_ref[...] = lax.rsqrt(deg_ref[0] + deg_ref[1]) * 0.7071067811865476


def _tc_rsqrt(degp):
    return pl.pallas_call(
        _rsqrt_body,
        out_shape=jax.ShapeDtypeStruct((2, 32768), F32),
        interpret=_INTERPRET,
    )(degp.reshape(2, 2, 32768))


_HIST_PER_TILE = E_EXT // 16
_HIST_CHUNK = 2176
_HIST_NCHUNK = _HIST_PER_TILE // _HIST_CHUNK


def _hist_body(src_hbm, dst_hbm, idx_hbm, dis2_hbm, hin_hbm, hout_hbm,
               idx_tab, dis_tab, ebuf_s, ebuf_d, fbuf_f, vbuf_f, fbuf2, vbuf2,
               zbuf, bounce, acc, sem):
    c = lax.axis_index("c")
    s = lax.axis_index("s")

    pltpu.sync_copy(idx_hbm, idx_tab)

    def fill_zero(i, carry):
        zbuf[pl.ds(i * 16, 16)] = jnp.zeros((16,), F32)
        return carry

    lax.fori_loop(0, 512, fill_zero, 0)

    lane = lax.broadcasted_iota(I32, (16,), 0)

    for p in range(8):
        direction = p // 4
        e8 = p % 4
        if e8 == 0:
            pltpu.sync_copy(dis2_hbm.at[pl.ds(direction * 32768, 32768)],
                            dis_tab)

        for bb in range(4):
            pltpu.sync_copy(zbuf, acc.at[pl.ds(s * 32768 + bb * 8192, 8192)])
        plsc.subcore_barrier()

        rng = e8 * 2 + c
        qbase = rng * Q

        def chunk(ch, carry):
            off = s * _HIST_PER_TILE + ch * _HIST_CHUNK
            pltpu.sync_copy(src_hbm.at[pl.ds(off, _HIST_CHUNK)], ebuf_s)
            pltpu.sync_copy(dst_hbm.at[pl.ds(off, _HIST_CHUNK)], ebuf_d)

            def group(i, cur):
                goff = i * 16
                s16 = ebuf_s[pl.ds(goff, 16)]
                d16 = ebuf_d[pl.ds(goff, 16)]
                a = plsc.load_gather(dis_tab, [s16])
                b2 = plsc.load_gather(dis_tab, [d16])
                val = a * b2
                if direction == 0:
                    rows, cfrom = d16, s16
                else:
                    rows, cfrom = s16, d16
                cls = plsc.load_gather(idx_tab, [cfrom])
                loc = rows - qbase
                ok = lax.bitcast_convert_type(loc, jnp.uint32) < jnp.uint32(Q)
                flat = (loc * 128 + cls) & (ACC - 1)
                plsc.store_compressed(fbuf_f.at[pl.ds(cur, 16)], flat, mask=ok)
                plsc.store_compressed(vbuf_f.at[pl.ds(cur, 16)], val, mask=ok)
                return cur + jnp.sum(ok.astype(I32))

            cursor = lax.fori_loop(0, _HIST_CHUNK // 16, group, jnp.int32(0))

            okall = lane < 16
            zval = jnp.zeros((16,), F32)

            def pad(k, carry2):
                po = cursor + k * 16
                plsc.store_compressed(fbuf_f.at[pl.ds(po, 16)],
                                      (lane + po) & (ACC - 1), mask=okall)
                plsc.store_compressed(vbuf_f.at[pl.ds(po, 16)], zval, mask=okall)
                return carry2

            lax.fori_loop(0, 8, pad, 0)
            nrows = (cursor + 127) >> 7

            def torow(k, carry2):
                fv = fbuf_f[pl.ds(k * 16, 16)]
                vv = vbuf_f[pl.ds(k * 16, 16)]
                jv = jnp.full((16,), 0, I32) + (k >> 3)
                cv = lane + (k & 7) * 16
                plsc.store_scatter(fbuf2, [jv, cv], fv)
                plsc.store_scatter(vbuf2, [jv, cv], vv)
                return carry2

            lax.fori_loop(0, nrows * 8, torow, 0)

            def fire(j, carry2):
                pltpu.sync_copy(vbuf2.at[j], acc.at[fbuf2.at[j]], add=True)
                return carry2

            lax.fori_loop(0, nrows, fire, 0)
            return carry

        lax.fori_loop(0, _HIST_NCHUNK, chunk, 0)
        plsc.subcore_barrier()

        out_ref = hin_hbm if direction == 0 else hout_hbm
        for bb in range(4):
            pltpu.sync_copy(acc.at[pl.ds(s * 32768 + bb * 8192, 8192)], bounce)
            pltpu.sync_copy(bounce,
                            out_ref.at[pl.ds(rng * ACC + s * 32768
                                             + bb * 8192, 8192)])
        plsc.subcore_barrier()


def _sc_hist(src_flat, dst_flat, idx, dis2):
    mesh = plsc.VectorSubcoreMesh(core_axis_name="c", subcore_axis_name="s")
    f = pl.kernel(
        _hist_body,
        out_type=[jax.ShapeDtypeStruct((N_TOTAL * D,), F32),
                  jax.ShapeDtypeStruct((N_TOTAL * D,), F32)],
        mesh=mesh,
        scratch_types=[
            pltpu.VMEM((N_TOTAL,), I32),
            pltpu.VMEM((N_TOTAL,), F32),
            pltpu.VMEM((_HIST_CHUNK,), I32),
            pltpu.VMEM((_HIST_CHUNK,), I32),
            pltpu.VMEM((_HIST_CHUNK + 128,), I32),
            pltpu.VMEM((_HIST_CHUNK + 128,), F32),
            pltpu.VMEM((18, 128), I32),
            pltpu.VMEM((18, 128), F32),
            pltpu.VMEM((8192,), F32),
            pltpu.VMEM((8192,), F32),
            pltpu.VMEM_SHARED((ACC,), F32),
            pltpu.SemaphoreType.DMA,
        ],
        compiler_params=pltpu.CompilerParams(needs_layout_passes=False),
    )
    return f(src_flat, dst_flat, idx, dis2)


_GB = 4
_RB = _GB * NUM_NODES


def _dense_body(idx_ref, hist_in_ref, hist_out_ref, emb_ref, Win_ref, Wout_ref,
                Wr_ref, BD1_ref, b1_ref, BD2_ref, b2_ref, bias_ref, ng_ref,
                nb_ref, We1_ref, be1_ref, eg_ref, eb_ref, We2r_ref, be2_ref,
                dots_ref, exit_ref, S_scr):
    b = pl.program_id(0)
    hi = lax.Precision.HIGHEST

    @pl.when(b == 0)
    def _():
        e = emb_ref[...]
        S_scr[0:D] = jnp.dot(e, Win_ref[...], preferred_element_type=F32,
                             precision=hi)
        S_scr[D:2 * D] = jnp.dot(e, Wout_ref[...], preferred_element_type=F32,
                                 precision=hi)
        S_scr[2 * D:3 * D] = jnp.dot(e, Wr_ref[...],
                                     preferred_element_type=F32, precision=hi)

    hp = lax.Precision.DEFAULT
    idxv = idx_ref[0]
    k_iota = lax.broadcasted_iota(I32, (D, _RB), 0)
    onehotT = (k_iota == idxv).astype(F32)

    xn = (jnp.dot(hist_in_ref[...], S_scr[0:D], preferred_element_type=F32,
                  precision=hp)
          + jnp.dot(hist_out_ref[...], S_scr[D:2 * D],
                    preferred_element_type=F32, precision=hp)
          + lax.dot_general(onehotT, S_scr[2 * D:3 * D],
                            (((0,), (0,)), ((), ())),
                            preferred_element_type=F32, precision=hp)
          + bias_ref[...])

    h1 = jnp.maximum(jnp.dot(xn, BD1_ref[...], preferred_element_type=F32,
                             precision=hp) + b1_ref[...], 0.0)
    x = jnp.dot(h1, BD2_ref[...], preferred_element_type=F32,
                precision=hp) + b2_ref[...]

    mu = jnp.mean(x, axis=1, keepdims=True)
    xc = x - mu
    var = jnp.mean(xc * xc, axis=1, keepdims=True)
    xln = xc * lax.rsqrt(var + 1e-5) * ng_ref[...] + nb_ref[...]

    m = jnp.concatenate(
        [jnp.mean(xln[g * NUM_NODES:(g + 1) * NUM_NODES], axis=0,
                  keepdims=True) for g in range(_GB)], axis=0)
    hm = jnp.maximum(jnp.dot(m, We1_ref[...], preferred_element_type=F32,
                             precision=hi) + be1_ref[...], 0.0)
    mu2 = jnp.mean(hm, axis=1, keepdims=True)
    hc = hm - mu2
    var2 = jnp.mean(hc * hc, axis=1, keepdims=True)
    hln = hc * lax.rsqrt(var2 + 1e-5) * eg_ref[...] + eb_ref[...]
    exit_ref[pl.ds(b * _GB, _GB), :] = (
        jnp.sum(hln * We2r_ref[...], axis=1, keepdims=True) + be2_ref[...])

    for g in range(_GB):
        blk = xln[g * NUM_NODES:(g + 1) * NUM_NODES]
        dots_ref[g * NUM_NODES:(g + 1) * NUM_NODES, :] = lax.dot_general(
            blk[:, :H], blk[:, H:], (((1,), (1,)), ((), ())),
            preferred_element_type=F32, precision=hp) * (1.0 / 8.0)


def _dense_pipeline(idx, hist_in, hist_out, emb, Win, Wout, Wr, BD1, b1, BD2,
                    b2, bias_comb, ng, nb, We1, be1, eg, eb, We2r, be2):
    full = lambda s: pl.BlockSpec(s, lambda b: tuple(0 for _ in s))
    dots, exitv = pl.pallas_call(
        _dense_body,
        grid=(B // _GB,),
        in_specs=[
            pl.BlockSpec((1, 1, _RB), lambda b: (b, 0, 0)),
            pl.BlockSpec((_RB, D), lambda b: (b, 0)),
            pl.BlockSpec((_RB, D), lambda b: (b, 0)),
            full((D, D)), full((D, D)), full((D, D)), full((D, D)),
            full((D, D)), full((1, D)), full((D, D)), full((1, D)),
            full((1, D)), full((1, D)), full((1, D)),
            full((D, D)), full((1, D)), full((1, D)), full((1, D)),
            full((1, D)), full((1, 1)),
        ],
        out_specs=[
            pl.BlockSpec((_RB, D), lambda b: (b, 0)),
            pl.BlockSpec((B, 1), lambda b: (0, 0)),
        ],
        out_shape=[
            jax.ShapeDtypeStruct((N_TOTAL, D), F32),
            jax.ShapeDtypeStruct((B, 1), F32),
        ],
        scratch_shapes=[pltpu.VMEM((3 * D, D), F32)],
        interpret=_INTERPRET,
    )(idx, hist_in, hist_out, emb, Win, Wout, Wr, BD1, b1, BD2, b2, bias_comb,
      ng, nb, We1, be1, eg, eb, We2r, be2)
    return dots, exitv


def kernel(node_features, edge_index, ptr, emb, Win, bin_, Wout, bout, Wr, br,
           Wi1, bi1, Wi2, bi2, Wo1, bo1, Wo2, bo2, ng, nb, We1, be1, eg, eb,
           We2, be2):
    idx = node_features.reshape(-1).astype(I32)
    src = edge_index[0].astype(I32)
    dst = edge_index[1].astype(I32)
    loop = jnp.arange(N_TOTAL, dtype=I32)
    src_ext = jnp.concatenate([src, loop])
    dst_ext = jnp.concatenate([dst, loop])

    degp = _sc_degrees(src_ext, dst_ext)
    dis2 = _tc_rsqrt(degp).reshape(-1)
    hin, hout = _sc_hist(src_ext, dst_ext, idx, dis2)
    hist_in = hin.reshape(N_TOTAL, D)
    hist_out = hout.reshape(N_TOTAL, D)

    BD1 = jnp.zeros((D, D), F32).at[:H, :H].set(Wi1).at[H:, H:].set(Wo1)
    BD2 = jnp.zeros((D, D), F32).at[:H, :H].set(Wi2).at[H:, H:].set(Wo2)
    b1 = jnp.concatenate([bi1, bo1]).reshape(1, D)
    b2 = jnp.concatenate([bi2, bo2]).reshape(1, D)
    bias_comb = (0.5 * bin_ + 0.5 * bout + br).reshape(1, D)

    dots, exitv = _dense_pipeline(
        idx.reshape(B // _GB, 1, _RB), hist_in, hist_out, emb, Win, Wout, Wr,
        BD1, b1, BD2, b2, bias_comb, ng.reshape(1, D), nb.reshape(1, D),
        We1, be1.reshape(1, D), eg.reshape(1, D), eb.reshape(1, D),
        We2.reshape(1, D), be2.reshape(1, 1))

    edge_actions = dots.reshape(B, NUM_NODES * NUM_NODES)
    return jnp.concatenate([edge_actions, exitv], axis=-1)


# speedup vs baseline: 59.5314x; 1.0007x over previous
import jax
import jax.numpy as jnp
from jax import lax
from jax.experimental import pallas as pl
from jax.experimental.pallas import tpu as pltpu
from jax.experimental.pallas import tpu_sc as plsc

N_TOTAL = 32768
B = 256
NUM_NODES = 128
D = 128
H = 64
F32 = jnp.float32
I32 = jnp.int32

E = 524288
E_EXT = E + N_TOTAL
Q = 4096
ACC = Q * D


_DEG_PER_TILE = E_EXT // 32


def _deg_body(src_hbm, dst_hbm, out_hbm, sbuf, dbuf, ones, zbuf, tbuf,
              acc_in, acc_out, sem):
    c = lax.axis_index("c")
    s = lax.axis_index("s")

    def fill_ones(i, carry):
        ones[pl.ds(i * 16, 16)] = jnp.full((16,), 1.0, F32)
        return carry

    lax.fori_loop(0, 8, fill_ones, 0)

    def fill_zero(i, carry):
        zbuf[pl.ds(i * 16, 16)] = jnp.zeros((16,), F32)
        return carry

    lax.fori_loop(0, 128, fill_zero, 0)

    pltpu.sync_copy(zbuf, acc_in.at[pl.ds(s * 2048, 2048)])
    pltpu.sync_copy(zbuf, acc_out.at[pl.ds(s * 2048, 2048)])
    plsc.subcore_barrier()

    tile_base = (c * 16 + s) * (_DEG_PER_TILE // 128)

    def chunk(ch, carry):
        off = tile_base + ch * 8
        pltpu.sync_copy(src_hbm.at[pl.ds(off, 8)], sbuf)
        pltpu.sync_copy(dst_hbm.at[pl.ds(off, 8)], dbuf)
        cps = []
        for j in range(8):
            cps.append(pltpu.async_copy(ones, acc_in.at[dbuf.at[j]], sem,
                                        add=True))
            cps.append(pltpu.async_copy(ones, acc_out.at[sbuf.at[j]], sem,
                                        add=True))
        for cp in cps:
            cp.wait()
        return carry

    lax.fori_loop(0, 17, chunk, 0)
    plsc.subcore_barrier()

    pltpu.sync_copy(acc_in.at[pl.ds(s * 2048, 2048)], tbuf)
    pltpu.sync_copy(tbuf, out_hbm.at[pl.ds(c * 65536 + s * 2048, 2048)])
    pltpu.sync_copy(acc_out.at[pl.ds(s * 2048, 2048)], tbuf)
    pltpu.sync_copy(tbuf, out_hbm.at[pl.ds(c * 65536 + 32768 + s * 2048, 2048)])


def _sc_degrees(src_flat, dst_flat):
    src2 = src_flat.reshape(E_EXT // 128, 128)
    dst2 = dst_flat.reshape(E_EXT // 128, 128)
    mesh = plsc.VectorSubcoreMesh(core_axis_name="c", subcore_axis_name="s")
    f = pl.kernel(
        _deg_body,
        out_type=jax.ShapeDtypeStruct((131072,), F32),
        mesh=mesh,
        scratch_types=[
            pltpu.VMEM((8, 128), I32),
            pltpu.VMEM((8, 128), I32),
            pltpu.VMEM((128,), F32),
            pltpu.VMEM((2048,), F32),
            pltpu.VMEM((2048,), F32),
            pltpu.VMEM_SHARED((32768,), F32),
            pltpu.VMEM_SHARED((32768,), F32),
            pltpu.SemaphoreType.DMA,
        ],
    )
    return f(src2, dst2)


def _rsqrt_body(deg_ref, out_ref):
    out_ref[...] = lax.rsqrt(deg_ref[0] + deg_ref[1]) * 0.7071067811865476


def _tc_rsqrt(degp):
    return pl.pallas_call(
        _rsqrt_body,
        out_shape=jax.ShapeDtypeStruct((2, 32768), F32),
    )(degp.reshape(2, 2, 32768))


_HIST_PER_TILE = E_EXT // 16
_HIST_CHUNK = 2176
_HIST_NCHUNK = _HIST_PER_TILE // _HIST_CHUNK


def _hist_body(src_hbm, dst_hbm, idx_hbm, dis2_hbm, hin_hbm, hout_hbm,
               idx_tab, dis_tab, ebuf_s, ebuf_d, fbuf_f, vbuf_f, fbuf2, vbuf2,
               zbuf, bounce, acc, sem):
    c = lax.axis_index("c")
    s = lax.axis_index("s")

    pltpu.sync_copy(idx_hbm, idx_tab)

    def fill_zero(i, carry):
        zbuf[pl.ds(i * 16, 16)] = jnp.zeros((16,), F32)
        return carry

    lax.fori_loop(0, 512, fill_zero, 0)

    lane = lax.broadcasted_iota(I32, (16,), 0)

    for p in range(8):
        direction = p // 4
        e8 = p % 4
        if e8 == 0:
            pltpu.sync_copy(dis2_hbm.at[pl.ds(direction * 32768, 32768)],
                            dis_tab)

        for bb in range(4):
            pltpu.sync_copy(zbuf, acc.at[pl.ds(s * 32768 + bb * 8192, 8192)])
        plsc.subcore_barrier()

        rng = e8 * 2 + c
        qbase = rng * Q

        def chunk(ch, carry):
            off = s * _HIST_PER_TILE + ch * _HIST_CHUNK
            pltpu.sync_copy(src_hbm.at[pl.ds(off, _HIST_CHUNK)], ebuf_s)
            pltpu.sync_copy(dst_hbm.at[pl.ds(off, _HIST_CHUNK)], ebuf_d)

            def one(goff, cur):
                s16 = ebuf_s[pl.ds(goff, 16)]
                d16 = ebuf_d[pl.ds(goff, 16)]
                a = plsc.load_gather(dis_tab, [s16])
                b2 = plsc.load_gather(dis_tab, [d16])
                val = a * b2
                if direction == 0:
                    rows, cfrom = d16, s16
                else:
                    rows, cfrom = s16, d16
                cls = plsc.load_gather(idx_tab, [cfrom])
                loc = rows - qbase
                ok = lax.bitcast_convert_type(loc, jnp.uint32) < jnp.uint32(Q)
                flat = (loc * 128 + cls) & (ACC - 1)
                plsc.store_compressed(fbuf_f.at[pl.ds(cur, 16)], flat, mask=ok)
                plsc.store_compressed(vbuf_f.at[pl.ds(cur, 16)], val, mask=ok)
                return cur + jnp.sum(ok.astype(I32))

            def group(i, cur):
                cur = one(i * 32, cur)
                return one(i * 32 + 16, cur)

            cursor = lax.fori_loop(0, _HIST_CHUNK // 32, group, jnp.int32(0))

            okall = lane < 16
            zval = jnp.zeros((16,), F32)

            def pad(k, carry2):
                po = cursor + k * 16
                plsc.store_compressed(fbuf_f.at[pl.ds(po, 16)],
                                      (lane + po) & (ACC - 1), mask=okall)
                plsc.store_compressed(vbuf_f.at[pl.ds(po, 16)], zval, mask=okall)
                return carry2

            lax.fori_loop(0, 8, pad, 0)
            nrows = (cursor + 127) >> 7

            def torow(k, carry2):
                fv = fbuf_f[pl.ds(k * 16, 16)]
                vv = vbuf_f[pl.ds(k * 16, 16)]
                jv = jnp.full((16,), 0, I32) + (k >> 3)
                cv = lane + (k & 7) * 16
                plsc.store_scatter(fbuf2, [jv, cv], fv)
                plsc.store_scatter(vbuf2, [jv, cv], vv)
                return carry2

            lax.fori_loop(0, nrows * 8, torow, 0)

            def fire(j, carry2):
                pltpu.sync_copy(vbuf2.at[j], acc.at[fbuf2.at[j]], add=True)
                return carry2

            lax.fori_loop(0, nrows, fire, 0)
            return carry

        lax.fori_loop(0, _HIST_NCHUNK, chunk, 0)
        plsc.subcore_barrier()

        out_ref = hin_hbm if direction == 0 else hout_hbm
        for bb in range(4):
            pltpu.sync_copy(acc.at[pl.ds(s * 32768 + bb * 8192, 8192)], bounce)
            pltpu.sync_copy(bounce,
                            out_ref.at[pl.ds(rng * ACC + s * 32768
                                             + bb * 8192, 8192)])
        plsc.subcore_barrier()


def _sc_hist(src_flat, dst_flat, idx, dis2):
    mesh = plsc.VectorSubcoreMesh(core_axis_name="c", subcore_axis_name="s")
    f = pl.kernel(
        _hist_body,
        out_type=[jax.ShapeDtypeStruct((N_TOTAL * D,), F32),
                  jax.ShapeDtypeStruct((N_TOTAL * D,), F32)],
        mesh=mesh,
        scratch_types=[
            pltpu.VMEM((N_TOTAL,), I32),
            pltpu.VMEM((N_TOTAL,), F32),
            pltpu.VMEM((_HIST_CHUNK,), I32),
            pltpu.VMEM((_HIST_CHUNK,), I32),
            pltpu.VMEM((_HIST_CHUNK + 128,), I32),
            pltpu.VMEM((_HIST_CHUNK + 128,), F32),
            pltpu.VMEM((18, 128), I32),
            pltpu.VMEM((18, 128), F32),
            pltpu.VMEM((8192,), F32),
            pltpu.VMEM((8192,), F32),
            pltpu.VMEM_SHARED((ACC,), F32),
            pltpu.SemaphoreType.DMA,
        ],
        compiler_params=pltpu.CompilerParams(needs_layout_passes=False),
    )
    return f(src_flat, dst_flat, idx, dis2)


_GB = 4
_RB = _GB * NUM_NODES


def _dense_body(idx_ref, hist_in_ref, hist_out_ref, emb_ref, Win_ref, Wout_ref,
                Wr_ref, BD1_ref, b1_ref, BD2_ref, b2_ref, bias_ref, ng_ref,
                nb_ref, We1_ref, be1_ref, eg_ref, eb_ref, We2r_ref, be2_ref,
                dots_ref, exit_ref, S_scr):
    b = pl.program_id(0)
    hi = lax.Precision.HIGHEST

    @pl.when(b == 0)
    def _():
        e = emb_ref[...]
        S_scr[0:D] = jnp.dot(e, Win_ref[...], preferred_element_type=F32,
                             precision=hi)
        S_scr[D:2 * D] = jnp.dot(e, Wout_ref[...], preferred_element_type=F32,
                                 precision=hi)
        S_scr[2 * D:3 * D] = jnp.dot(e, Wr_ref[...],
                                     preferred_element_type=F32, precision=hi)

    hp = lax.Precision.DEFAULT
    idxv = idx_ref[0]
    k_iota = lax.broadcasted_iota(I32, (D, _RB), 0)
    onehotT = (k_iota == idxv).astype(F32)

    xn = (jnp.dot(hist_in_ref[...], S_scr[0:D], preferred_element_type=F32,
                  precision=hp)
          + jnp.dot(hist_out_ref[...], S_scr[D:2 * D],
                    preferred_element_type=F32, precision=hp)
          + lax.dot_general(onehotT, S_scr[2 * D:3 * D],
                            (((0,), (0,)), ((), ())),
                            preferred_element_type=F32, precision=hp)
          + bias_ref[...])

    h1 = jnp.maximum(jnp.dot(xn, BD1_ref[...], preferred_element_type=F32,
                             precision=hp) + b1_ref[...], 0.0)
    x = jnp.dot(h1, BD2_ref[...], preferred_element_type=F32,
                precision=hp) + b2_ref[...]

    mu = jnp.mean(x, axis=1, keepdims=True)
    xc = x - mu
    var = jnp.mean(xc * xc, axis=1, keepdims=True)
    xln = xc * lax.rsqrt(var + 1e-5) * ng_ref[...] + nb_ref[...]

    m = jnp.concatenate(
        [jnp.mean(xln[g * NUM_NODES:(g + 1) * NUM_NODES], axis=0,
                  keepdims=True) for g in range(_GB)], axis=0)
    hm = jnp.maximum(jnp.dot(m, We1_ref[...], preferred_element_type=F32,
                             precision=hi) + be1_ref[...], 0.0)
    mu2 = jnp.mean(hm, axis=1, keepdims=True)
    hc = hm - mu2
    var2 = jnp.mean(hc * hc, axis=1, keepdims=True)
    hln = hc * lax.rsqrt(var2 + 1e-5) * eg_ref[...] + eb_ref[...]
    exit_ref[pl.ds(b * _GB, _GB), :] = (
        jnp.sum(hln * We2r_ref[...], axis=1, keepdims=True) + be2_ref[...])

    for g in range(_GB):
        blk = xln[g * NUM_NODES:(g + 1) * NUM_NODES]
        dots_ref[g * NUM_NODES:(g + 1) * NUM_NODES, :] = lax.dot_general(
            blk[:, :H], blk[:, H:], (((1,), (1,)), ((), ())),
            preferred_element_type=F32, precision=hp) * (1.0 / 8.0)


def _dense_pipeline(idx, hist_in, hist_out, emb, Win, Wout, Wr, BD1, b1, BD2,
                    b2, bias_comb, ng, nb, We1, be1, eg, eb, We2r, be2):
    full = lambda s: pl.BlockSpec(s, lambda b: tuple(0 for _ in s))
    dots, exitv = pl.pallas_call(
        _dense_body,
        grid=(B // _GB,),
        in_specs=[
            pl.BlockSpec((1, 1, _RB), lambda b: (b, 0, 0)),
            pl.BlockSpec((_RB, D), lambda b: (b, 0)),
            pl.BlockSpec((_RB, D), lambda b: (b, 0)),
            full((D, D)), full((D, D)), full((D, D)), full((D, D)),
            full((D, D)), full((1, D)), full((D, D)), full((1, D)),
            full((1, D)), full((1, D)), full((1, D)),
            full((D, D)), full((1, D)), full((1, D)), full((1, D)),
            full((1, D)), full((1, 1)),
        ],
        out_specs=[
            pl.BlockSpec((_RB, D), lambda b: (b, 0)),
            pl.BlockSpec((B, 1), lambda b: (0, 0)),
        ],
        out_shape=[
            jax.ShapeDtypeStruct((N_TOTAL, D), F32),
            jax.ShapeDtypeStruct((B, 1), F32),
        ],
        scratch_shapes=[pltpu.VMEM((3 * D, D), F32)],
    )(idx, hist_in, hist_out, emb, Win, Wout, Wr, BD1, b1, BD2, b2, bias_comb,
      ng, nb, We1, be1, eg, eb, We2r, be2)
    return dots, exitv


def kernel(node_features, edge_index, ptr, emb, Win, bin_, Wout, bout, Wr, br,
           Wi1, bi1, Wi2, bi2, Wo1, bo1, Wo2, bo2, ng, nb, We1, be1, eg, eb,
           We2, be2):
    idx = node_features.reshape(-1).astype(I32)
    src = edge_index[0].astype(I32)
    dst = edge_index[1].astype(I32)
    loop = jnp.arange(N_TOTAL, dtype=I32)
    src_ext = jnp.concatenate([src, loop])
    dst_ext = jnp.concatenate([dst, loop])

    degp = _sc_degrees(src_ext, dst_ext)
    dis2 = _tc_rsqrt(degp).reshape(-1)
    hin, hout = _sc_hist(src_ext, dst_ext, idx, dis2)
    hist_in = hin.reshape(N_TOTAL, D)
    hist_out = hout.reshape(N_TOTAL, D)

    BD1 = jnp.zeros((D, D), F32).at[:H, :H].set(Wi1).at[H:, H:].set(Wo1)
    BD2 = jnp.zeros((D, D), F32).at[:H, :H].set(Wi2).at[H:, H:].set(Wo2)
    b1 = jnp.concatenate([bi1, bo1]).reshape(1, D)
    b2 = jnp.concatenate([bi2, bo2]).reshape(1, D)
    bias_comb = (0.5 * bin_ + 0.5 * bout + br).reshape(1, D)

    dots, exitv = _dense_pipeline(
        idx.reshape(B // _GB, 1, _RB), hist_in, hist_out, emb, Win, Wout, Wr,
        BD1, b1, BD2, b2, bias_comb, ng.reshape(1, D), nb.reshape(1, D),
        We1, be1.reshape(1, D), eg.reshape(1, D), eb.reshape(1, D),
        We2.reshape(1, D), be2.reshape(1, 1))

    edge_actions = dots.reshape(B, NUM_NODES * NUM_NODES)
    return jnp.concatenate([edge_actions, exitv], axis=-1)
